# Initial kernel scaffold; baseline (speedup 1.0000x reference)
import jax
import jax.numpy as jnp
from jax.experimental import pallas as pl


def kernel(coord_v):
    raise NotImplementedError("write your pallas kernel here")


# speedup vs baseline: 4.8586x; 4.8586x over previous
import functools

import jax
import jax.numpy as jnp
import numpy as np
from jax import lax
from jax.experimental import pallas as pl
from jax.experimental.pallas import tpu as pltpu
from jax.experimental.pallas import tpu_sc as plsc

N_BATCH = 1024
N_PTS = 64
NMC = 8
L = 32
HALF = NMC * L * L
SLAB = 2 * HALF
NW = 32
B_PER_W = N_BATCH // NW
CHUNKS = N_PTS // 16


def _log10_16(x):
    bits = lax.bitcast_convert_type(x, jnp.int32)
    e = lax.shift_right_logical(bits, 23) - 127
    f = lax.bitcast_convert_type(
        (bits & 0x007FFFFF) | 0x3F800000, jnp.float32)
    big = f > np.float32(1.4142135)
    f = jnp.where(big, f * np.float32(0.5), f)
    ef = (e + big.astype(jnp.int32)).astype(jnp.float32)
    s = (f - 1.0) / (f + 1.0)
    s2 = s * s
    lnf = 2.0 * s * (1.0 + s2 * (np.float32(1 / 3) + s2 * (
        np.float32(1 / 5) + s2 * (np.float32(1 / 7) + s2 * np.float32(1 / 9)))))
    return ef * np.float32(0.30103001) + lnf * np.float32(0.43429449)


def _body(coord_hbm, out_hbm, coords, slabs, idxsaves, sems):
    c_ax = lax.axis_index("c")
    s_ax = lax.axis_index("s")
    wid = s_ax * 2 + c_ax
    base_b = wid * B_PER_W

    one16 = jnp.full((16,), 1.0, jnp.float32)
    zero16 = jnp.zeros((16,), jnp.float32)
    iota = lax.iota(jnp.int32, 16)
    sel0 = iota * 3

    coord_cp = pltpu.async_copy(
        coord_hbm.at[pl.ds(base_b * 3 * N_PTS, B_PER_W * 3 * N_PTS)],
        coords, sems[0])

    def fill(i, _):
        off = pl.multiple_of(i * 16, 16)
        for slab in slabs:
            slab[pl.ds(off, 16)] = one16
            slab[pl.ds(HALF + off, 16)] = zero16
        return _
    lax.fori_loop(0, HALF // 16, fill, None)

    coord_cp.wait()

    def point_indices(g, c):
        sel = sel0 + (g * 3 * N_PTS + c * 48)
        x = plsc.load_gather(coords, [sel])
        y = plsc.load_gather(coords, [sel + 1])
        m = plsc.load_gather(coords, [sel + 2])
        xi = (x * np.float32(L)).astype(jnp.int32)
        yi = (y * np.float32(L)).astype(jnp.int32)
        mi = (_log10_16(m) * np.float32(4.0)).astype(jnp.int32)
        return mi * (L * L) + yi * L + xi

    pending = [None, None]
    for g in range(B_PER_W):
        p = g % 2
        slab, idxsave = slabs[p], idxsaves[p]
        if pending[p] is not None:
            pending[p].wait()
            for c in range(CHUNKS):
                f0 = idxsave[pl.ds(c * 16, 16)]
                plsc.store_scatter(slab, [f0], one16)
                plsc.store_scatter(slab, [f0 + HALF], zero16)
        for c in range(CHUNKS):
            f0 = point_indices(g, c)
            plsc.store_scatter(slab, [f0], zero16)
            plsc.store_scatter(slab, [f0 + HALF], one16)
            idxsave[pl.ds(c * 16, 16)] = f0
        cp = pltpu.async_copy(slab, out_hbm.at[base_b + g], sems[p])
        pending[p] = cp
    for cp in pending:
        cp.wait()


@functools.partial(jax.jit, static_argnames=())
def kernel(coord_v):
    mesh = plsc.VectorSubcoreMesh(core_axis_name="c", subcore_axis_name="s")
    out2d = pl.kernel(
        lambda coord_hbm, out_hbm, coords, slabA, slabB, idxA, idxB, semA, semB:
            _body(coord_hbm, out_hbm, coords, (slabA, slabB), (idxA, idxB),
                  (semA, semB)),
        out_type=jax.ShapeDtypeStruct((N_BATCH, SLAB), jnp.float32),
        mesh=mesh,
        compiler_params=pltpu.CompilerParams(needs_layout_passes=False),
        scratch_types=[
            pltpu.VMEM((B_PER_W * 3 * N_PTS,), jnp.float32),
            pltpu.VMEM((SLAB,), jnp.float32),
            pltpu.VMEM((SLAB,), jnp.float32),
            pltpu.VMEM((N_PTS,), jnp.int32),
            pltpu.VMEM((N_PTS,), jnp.int32),
            pltpu.SemaphoreType.DMA,
            pltpu.SemaphoreType.DMA,
        ],
    )(coord_v.reshape(-1))
    return out2d.reshape(N_BATCH, 2 * NMC, L, L)
